# baseline (speedup 1.0000x reference)
import functools

import jax
import jax.numpy as jnp
from jax import lax
from jax.experimental import pallas as pl
from jax.experimental.pallas import tpu as pltpu
from jax.experimental.pallas import tpu_sc as plsc

B, C, H, W = 256, 256, 28, 28
GH, GW = 14, 14
NPTS = GH * GW
IMG = H * W
L = 16
NGRP = (NPTS + L - 1) // L
NCORES, NSUB = 2, 16
NW = NCORES * NSUB
BPW = B // NW
CCH = 64
NCHUNK = C // CCH
IMG_CH = CCH * IMG
OUT_CH = CCH * NPTS


def _tap_setup(grid_v, idx_v, wt_v):
    for g in range(NGRP):
        p = lax.iota(jnp.int32, L) + (g * L)
        p = jnp.minimum(p, NPTS - 1)
        gy = plsc.load_gather(grid_v, [p * 2])
        gx = plsc.load_gather(grid_v, [p * 2 + 1])
        y = (gy + 1.0) * ((H - 1) * 0.5)
        x = (gx + 1.0) * ((W - 1) * 0.5)
        y = jnp.minimum(jnp.maximum(y, 0.0), float(H - 1))
        x = jnp.minimum(jnp.maximum(x, 0.0), float(W - 1))
        y0 = y.astype(jnp.int32)
        x0 = x.astype(jnp.int32)
        wy1 = y - y0.astype(jnp.float32)
        wx1 = x - x0.astype(jnp.float32)
        y1 = jnp.minimum(y0 + 1, H - 1)
        x1 = jnp.minimum(x0 + 1, W - 1)
        yb0 = y0 * W
        yb1 = y1 * W
        wy0 = 1.0 - wy1
        wx0 = 1.0 - wx1
        sl = pl.ds(g * L, L)
        idx_v[0, sl] = yb0 + x0
        idx_v[1, sl] = yb0 + x1
        idx_v[2, sl] = yb1 + x0
        idx_v[3, sl] = yb1 + x1
        wt_v[0, sl] = wy0 * wx0
        wt_v[1, sl] = wy0 * wx1
        wt_v[2, sl] = wy1 * wx0
        wt_v[3, sl] = wy1 * wx1


def _body(img_hbm, grid_hbm, out_hbm, grid_v, idx_v, wt_v, img_v0, img_v1,
          out_v0, out_v1, gsem, isem0, isem1, osem0, osem1):
    cid = lax.axis_index("c")
    sid = lax.axis_index("s")
    wid = sid * NCORES + cid
    imgs = (img_v0, img_v1)
    outs = (out_v0, out_v1)
    isems = (isem0, isem1)
    osems = (osem0, osem1)

    def img_copy(b, k, buf):
        base = (b * C + k * CCH) * IMG
        return pltpu.make_async_copy(
            img_hbm.at[pl.ds(base, IMG_CH)], imgs[buf], isems[buf])

    def out_copy(b, k, buf):
        base = (b * C + k * CCH) * NPTS
        return pltpu.make_async_copy(
            outs[buf], out_hbm.at[pl.ds(base, OUT_CH)], osems[buf])

    def batch_body(bi, carry):
        b = wid * BPW + bi
        gcp = pltpu.make_async_copy(
            grid_hbm.at[pl.ds(b * (NPTS * 2), NPTS * 2)], grid_v, gsem)
        gcp.start()
        img_copy(b, 0, 0).start()
        gcp.wait()
        _tap_setup(grid_v, idx_v, wt_v)

        for k in range(NCHUNK):
            buf = k % 2
            if k + 1 < NCHUNK:
                img_copy(b, k + 1, 1 - buf).start()
            img_copy(b, k, buf).wait()
            if k >= 2:
                out_copy(b, k - 2, buf).wait()
            img_flat = imgs[buf]
            out_flat = outs[buf]
            for g in range(NGRP):
                sl = pl.ds(g * L, L)
                i00 = idx_v[0, sl]
                i01 = idx_v[1, sl]
                i10 = idx_v[2, sl]
                i11 = idx_v[3, sl]
                w00 = wt_v[0, sl]
                w01 = wt_v[1, sl]
                w10 = wt_v[2, sl]
                w11 = wt_v[3, sl]
                og = lax.iota(jnp.int32, L) + (g * L)
                mask = None
                if (g + 1) * L > NPTS:
                    mask = og < NPTS

                def ch_body(c, acc):
                    cb = jnp.full((L,), c * IMG, jnp.int32)
                    ob = jnp.full((L,), c * NPTS, jnp.int32) + og
                    v00 = plsc.load_gather(img_flat, [i00 + cb])
                    v01 = plsc.load_gather(img_flat, [i01 + cb])
                    v10 = plsc.load_gather(img_flat, [i10 + cb])
                    v11 = plsc.load_gather(img_flat, [i11 + cb])
                    res = v00 * w00 + v01 * w01 + v10 * w10 + v11 * w11
                    plsc.store_scatter(out_flat, [ob], res, mask=mask)
                    return acc

                lax.fori_loop(0, CCH, ch_body, 0)
            out_copy(b, k, buf).start()

        out_copy(b, NCHUNK - 2, 0).wait()
        out_copy(b, NCHUNK - 1, 1).wait()
        return carry

    lax.fori_loop(0, BPW, batch_body, 0)


@jax.jit
def _roicrop_sc(img_flat, grid_flat):
    mesh = plsc.VectorSubcoreMesh(
        core_axis_name="c", subcore_axis_name="s",
        num_cores=NCORES, num_subcores=NSUB)
    return pl.kernel(
        _body,
        out_type=jax.ShapeDtypeStruct((B * C * NPTS,), jnp.float32),
        mesh=mesh,
        compiler_params=pltpu.CompilerParams(needs_layout_passes=False),
        scratch_types=[
            pltpu.VMEM((NPTS * 2,), jnp.float32),
            pltpu.VMEM((4, NGRP * L), jnp.int32),
            pltpu.VMEM((4, NGRP * L), jnp.float32),
            pltpu.VMEM((IMG_CH,), jnp.float32),
            pltpu.VMEM((IMG_CH,), jnp.float32),
            pltpu.VMEM((OUT_CH,), jnp.float32),
            pltpu.VMEM((OUT_CH,), jnp.float32),
            pltpu.SemaphoreType.DMA,
            pltpu.SemaphoreType.DMA,
            pltpu.SemaphoreType.DMA,
            pltpu.SemaphoreType.DMA,
            pltpu.SemaphoreType.DMA,
        ],
    )(img_flat, grid_flat)


def kernel(input1, input2):
    out = _roicrop_sc(input1.reshape(-1), input2.reshape(-1))
    return out.reshape(B, C, GH, GW)


# speedup vs baseline: 1.1004x; 1.1004x over previous
import functools

import jax
import jax.numpy as jnp
from jax import lax
from jax.experimental import pallas as pl
from jax.experimental.pallas import tpu as pltpu
from jax.experimental.pallas import tpu_sc as plsc

B, C, H, W = 256, 256, 28, 28
GH, GW = 14, 14
NPTS = GH * GW
IMG = H * W
L = 16
NGRP = (NPTS + L - 1) // L
NCORES, NSUB = 2, 16
NW = NCORES * NSUB
BPW = B // NW
CCH = 64
NCHUNK = C // CCH
IMG_CH = CCH * IMG
OUT_CH = CCH * NPTS


def _tap_setup(grid_v, idx_v, wt_v):
    for g in range(NGRP):
        p = lax.iota(jnp.int32, L) + (g * L)
        p = jnp.minimum(p, NPTS - 1)
        gy = plsc.load_gather(grid_v, [p * 2])
        gx = plsc.load_gather(grid_v, [p * 2 + 1])
        y = (gy + 1.0) * ((H - 1) * 0.5)
        x = (gx + 1.0) * ((W - 1) * 0.5)
        y = jnp.minimum(jnp.maximum(y, 0.0), float(H - 1))
        x = jnp.minimum(jnp.maximum(x, 0.0), float(W - 1))
        y0 = y.astype(jnp.int32)
        x0 = x.astype(jnp.int32)
        wy1 = y - y0.astype(jnp.float32)
        wx1 = x - x0.astype(jnp.float32)
        y1 = jnp.minimum(y0 + 1, H - 1)
        x1 = jnp.minimum(x0 + 1, W - 1)
        yb0 = y0 * W
        yb1 = y1 * W
        wy0 = 1.0 - wy1
        wx0 = 1.0 - wx1
        sl = pl.ds(g * L, L)
        idx_v[0, sl] = yb0 + x0
        idx_v[1, sl] = yb0 + x1
        idx_v[2, sl] = yb1 + x0
        idx_v[3, sl] = yb1 + x1
        wt_v[0, sl] = wy0 * wx0
        wt_v[1, sl] = wy0 * wx1
        wt_v[2, sl] = wy1 * wx0
        wt_v[3, sl] = wy1 * wx1


def _body(img_hbm, grid_hbm, out_hbm, grid_v, idx_v, wt_v, img_v0, img_v1,
          out_v0, out_v1, gsem, isem0, isem1, osem0, osem1):
    cid = lax.axis_index("c")
    sid = lax.axis_index("s")
    wid = sid * NCORES + cid
    imgs = (img_v0, img_v1)
    outs = (out_v0, out_v1)
    isems = (isem0, isem1)
    osems = (osem0, osem1)

    def img_copy(b, k, buf):
        base = (b * C + k * CCH) * IMG
        return pltpu.make_async_copy(
            img_hbm.at[pl.ds(base, IMG_CH)], imgs[buf], isems[buf])

    def out_copy(b, k, buf):
        base = (b * C + k * CCH) * NPTS
        return pltpu.make_async_copy(
            outs[buf], out_hbm.at[pl.ds(base, OUT_CH)], osems[buf])

    def batch_body(bi, carry):
        b = wid * BPW + bi
        gcp = pltpu.make_async_copy(
            grid_hbm.at[pl.ds(b * (NPTS * 2), NPTS * 2)], grid_v, gsem)
        gcp.start()
        img_copy(b, 0, 0).start()
        gcp.wait()
        _tap_setup(grid_v, idx_v, wt_v)

        for k in range(NCHUNK):
            buf = k % 2
            if k + 1 < NCHUNK:
                img_copy(b, k + 1, 1 - buf).start()
            img_copy(b, k, buf).wait()
            if k >= 2:
                out_copy(b, k - 2, buf).wait()
            img_flat = imgs[buf]
            out_flat = outs[buf]
            for g in range(NGRP):
                sl = pl.ds(g * L, L)
                i00 = idx_v[0, sl]
                i01 = idx_v[1, sl]
                i10 = idx_v[2, sl]
                i11 = idx_v[3, sl]
                w00 = wt_v[0, sl]
                w01 = wt_v[1, sl]
                w10 = wt_v[2, sl]
                w11 = wt_v[3, sl]
                og = lax.iota(jnp.int32, L) + (g * L)
                mask = None
                if (g + 1) * L > NPTS:
                    mask = og < NPTS

                @plsc.parallel_loop(0, CCH, unroll=4)
                def _ch_body(c):
                    cb = jnp.full((L,), c * IMG, jnp.int32)
                    ob = jnp.full((L,), c * NPTS, jnp.int32) + og
                    v00 = plsc.load_gather(img_flat, [i00 + cb])
                    v01 = plsc.load_gather(img_flat, [i01 + cb])
                    v10 = plsc.load_gather(img_flat, [i10 + cb])
                    v11 = plsc.load_gather(img_flat, [i11 + cb])
                    res = v00 * w00 + v01 * w01 + v10 * w10 + v11 * w11
                    plsc.store_scatter(out_flat, [ob], res, mask=mask)
            out_copy(b, k, buf).start()

        out_copy(b, NCHUNK - 2, 0).wait()
        out_copy(b, NCHUNK - 1, 1).wait()
        return carry

    lax.fori_loop(0, BPW, batch_body, 0)


@jax.jit
def _roicrop_sc(img_flat, grid_flat):
    mesh = plsc.VectorSubcoreMesh(
        core_axis_name="c", subcore_axis_name="s",
        num_cores=NCORES, num_subcores=NSUB)
    return pl.kernel(
        _body,
        out_type=jax.ShapeDtypeStruct((B * C * NPTS,), jnp.float32),
        mesh=mesh,
        compiler_params=pltpu.CompilerParams(needs_layout_passes=False),
        scratch_types=[
            pltpu.VMEM((NPTS * 2,), jnp.float32),
            pltpu.VMEM((4, NGRP * L), jnp.int32),
            pltpu.VMEM((4, NGRP * L), jnp.float32),
            pltpu.VMEM((IMG_CH,), jnp.float32),
            pltpu.VMEM((IMG_CH,), jnp.float32),
            pltpu.VMEM((OUT_CH,), jnp.float32),
            pltpu.VMEM((OUT_CH,), jnp.float32),
            pltpu.SemaphoreType.DMA,
            pltpu.SemaphoreType.DMA,
            pltpu.SemaphoreType.DMA,
            pltpu.SemaphoreType.DMA,
            pltpu.SemaphoreType.DMA,
        ],
    )(img_flat, grid_flat)


def kernel(input1, input2):
    out = _roicrop_sc(input1.reshape(-1), input2.reshape(-1))
    return out.reshape(B, C, GH, GW)


# speedup vs baseline: 1.3073x; 1.1880x over previous
import functools

import jax
import jax.numpy as jnp
from jax import lax
from jax.experimental import pallas as pl
from jax.experimental.pallas import tpu as pltpu
from jax.experimental.pallas import tpu_sc as plsc

B, C, H, W = 256, 256, 28, 28
GH, GW = 14, 14
NPTS = GH * GW
L = 16
NGRP = (NPTS + L - 1) // L
NCORES, NSUB = 2, 16
NW = NCORES * NSUB
BPW = B // NW
CCH = 8
NCHUNK = C // CCH


def _tap_setup(grid_v, idx_v, wt_v):
    for g in range(NGRP):
        sl = pl.ds(g * L, L)
        p = lax.iota(jnp.int32, L) + (g * L)
        p = jnp.minimum(p, NPTS - 1)
        gy = grid_v[0, sl]
        gx = grid_v[1, sl]
        y = (gy + 1.0) * ((H - 1) * 0.5)
        x = (gx + 1.0) * ((W - 1) * 0.5)
        y = jnp.minimum(jnp.maximum(y, 0.0), float(H - 1))
        x = jnp.minimum(jnp.maximum(x, 0.0), float(W - 1))
        y0 = y.astype(jnp.int32)
        x0 = x.astype(jnp.int32)
        wy1 = y - y0.astype(jnp.float32)
        wx1 = x - x0.astype(jnp.float32)
        y1 = jnp.minimum(y0 + 1, H - 1)
        x1 = jnp.minimum(x0 + 1, W - 1)
        wy0 = 1.0 - wy1
        wx0 = 1.0 - wx1
        idx_v[0, sl] = y0
        idx_v[1, sl] = y1
        idx_v[2, sl] = x0
        idx_v[3, sl] = x1
        idx_v[4, sl] = lax.div(p, GW)
        idx_v[5, sl] = lax.rem(p, GW)
        wt_v[0, sl] = wy0 * wx0
        wt_v[1, sl] = wy0 * wx1
        wt_v[2, sl] = wy1 * wx0
        wt_v[3, sl] = wy1 * wx1


def _body(img_hbm, grid_hbm, out_hbm, grid_v, idx_v, wt_v, img_v0, img_v1,
          out_v0, out_v1, gsem, isem0, isem1, osem0, osem1):
    cid = lax.axis_index("c")
    sid = lax.axis_index("s")
    wid = sid * NCORES + cid
    imgs = (img_v0, img_v1)
    outs = (out_v0, out_v1)
    isems = (isem0, isem1)
    osems = (osem0, osem1)

    def img_copy(b, k, buf):
        src = img_hbm.at[b, pl.ds(k * CCH, CCH)]
        return pltpu.make_async_copy(src, imgs[buf].reshape(CCH, H, W),
                                     isems[buf])

    def out_copy(b, k, buf):
        dst = out_hbm.at[b, pl.ds(k * CCH, CCH)]
        return pltpu.make_async_copy(outs[buf].reshape(CCH, GH, GW), dst,
                                     osems[buf])

    def do_chunk(b, buf):
        img_ref = imgs[buf]
        out_ref = outs[buf]
        for g in range(NGRP):
            sl = pl.ds(g * L, L)
            y0 = idx_v[0, sl]
            y1 = idx_v[1, sl]
            x0 = idx_v[2, sl]
            x1 = idx_v[3, sl]
            oyr = idx_v[4, sl]
            ox = idx_v[5, sl]
            w00 = wt_v[0, sl]
            w01 = wt_v[1, sl]
            w10 = wt_v[2, sl]
            w11 = wt_v[3, sl]
            mask = None
            if (g + 1) * L > NPTS:
                mask = (lax.iota(jnp.int32, L) + g * L) < NPTS
            for c in range(CCH):
                rb = c * H
                ob = c * GH
                v00 = plsc.load_gather(img_ref, [y0 + rb, x0])
                v01 = plsc.load_gather(img_ref, [y0 + rb, x1])
                v10 = plsc.load_gather(img_ref, [y1 + rb, x0])
                v11 = plsc.load_gather(img_ref, [y1 + rb, x1])
                res = v00 * w00 + v01 * w01 + v10 * w10 + v11 * w11
                plsc.store_scatter(out_ref, [oyr + ob, ox], res, mask=mask)

    def batch_body(bi, carry):
        b = wid * BPW + bi
        gcp = pltpu.make_async_copy(grid_hbm.at[b], grid_v, gsem)
        gcp.start()
        img_copy(b, 0, 0).start()
        gcp.wait()
        _tap_setup(grid_v, idx_v, wt_v)

        def chunk_pair(j, carry2):
            k0 = j * 2
            img_copy(b, k0 + 1, 1).start()

            @pl.when(j > 0)
            def _():
                out_copy(b, k0 - 2, 0).wait()

            img_copy(b, k0, 0).wait()
            do_chunk(b, 0)
            out_copy(b, k0, 0).start()
            @pl.when(j + 1 < NCHUNK // 2)
            def _():
                img_copy(b, k0 + 2, 0).start()

            @pl.when(j > 0)
            def _():
                out_copy(b, k0 - 1, 1).wait()

            img_copy(b, k0 + 1, 1).wait()
            do_chunk(b, 1)
            out_copy(b, k0 + 1, 1).start()
            return carry2

        lax.fori_loop(0, NCHUNK // 2, chunk_pair, 0)
        out_copy(b, NCHUNK - 2, 0).wait()
        out_copy(b, NCHUNK - 1, 1).wait()
        return carry

    lax.fori_loop(0, BPW, batch_body, 0)


@jax.jit
def kernel(input1, input2):
    mesh = plsc.VectorSubcoreMesh(
        core_axis_name="c", subcore_axis_name="s",
        num_cores=NCORES, num_subcores=NSUB)
    grid = jnp.transpose(input2.reshape(B, NPTS, 2), (0, 2, 1))
    grid = jnp.pad(grid, ((0, 0), (0, 0), (0, NGRP * L - NPTS)))
    return pl.kernel(
        _body,
        out_type=jax.ShapeDtypeStruct((B, C, GH, GW), jnp.float32),
        mesh=mesh,
        compiler_params=pltpu.CompilerParams(needs_layout_passes=False),
        scratch_types=[
            pltpu.VMEM((2, NGRP * L), jnp.float32),
            pltpu.VMEM((6, NGRP * L), jnp.int32),
            pltpu.VMEM((4, NGRP * L), jnp.float32),
            pltpu.VMEM((CCH * H, W), jnp.float32),
            pltpu.VMEM((CCH * H, W), jnp.float32),
            pltpu.VMEM((CCH * GH, GW), jnp.float32),
            pltpu.VMEM((CCH * GH, GW), jnp.float32),
            pltpu.SemaphoreType.DMA,
            pltpu.SemaphoreType.DMA,
            pltpu.SemaphoreType.DMA,
            pltpu.SemaphoreType.DMA,
            pltpu.SemaphoreType.DMA,
        ],
    )(input1, grid)


# speedup vs baseline: 1.3455x; 1.0292x over previous
import functools

import jax
import jax.numpy as jnp
from jax import lax
from jax.experimental import pallas as pl
from jax.experimental.pallas import tpu as pltpu
from jax.experimental.pallas import tpu_sc as plsc

B, C, H, W = 256, 256, 28, 28
GH, GW = 14, 14
NPTS = GH * GW
L = 16
NGRP = (NPTS + L - 1) // L
NCORES, NSUB = 2, 16
NW = NCORES * NSUB
BPW = B // NW
CCH = 8
NCHUNK = C // CCH


def _tap_setup(grid_v, idx_v, wt_v):
    for g in range(NGRP):
        sl = pl.ds(g * L, L)
        p = lax.iota(jnp.int32, L) + (g * L)
        p = jnp.minimum(p, NPTS - 1)
        gy = grid_v[0, sl]
        gx = grid_v[1, sl]
        y = (gy + 1.0) * ((H - 1) * 0.5)
        x = (gx + 1.0) * ((W - 1) * 0.5)
        y = jnp.minimum(jnp.maximum(y, 0.0), float(H - 1))
        x = jnp.minimum(jnp.maximum(x, 0.0), float(W - 1))
        y0 = y.astype(jnp.int32)
        x0 = x.astype(jnp.int32)
        wy1 = y - y0.astype(jnp.float32)
        wx1 = x - x0.astype(jnp.float32)
        y1 = jnp.minimum(y0 + 1, H - 1)
        x1 = jnp.minimum(x0 + 1, W - 1)
        wy0 = 1.0 - wy1
        wx0 = 1.0 - wx1
        yb0 = y0 * 128
        yb1 = y1 * 128
        idx_v[0, sl] = yb0 + x0
        idx_v[1, sl] = yb0 + x1
        idx_v[2, sl] = yb1 + x0
        idx_v[3, sl] = yb1 + x1
        idx_v[4, sl] = lax.div(p, GW) * 128 + lax.rem(p, GW)
        wt_v[0, sl] = wy0 * wx0
        wt_v[1, sl] = wy0 * wx1
        wt_v[2, sl] = wy1 * wx0
        wt_v[3, sl] = wy1 * wx1


def _body(img_hbm, grid_hbm, out_hbm, grid_v, idx_v, wt_v, img_v0, img_v1,
          out_v0, out_v1, gsem, isem0, isem1, osem0, osem1):
    cid = lax.axis_index("c")
    sid = lax.axis_index("s")
    wid = sid * NCORES + cid
    imgs = (img_v0, img_v1)
    outs = (out_v0, out_v1)
    isems = (isem0, isem1)
    osems = (osem0, osem1)

    def img_copy(b, k, buf):
        src = img_hbm.at[b, pl.ds(k * CCH, CCH)]
        return pltpu.make_async_copy(src, imgs[buf].reshape(CCH, H, W),
                                     isems[buf])

    def out_copy(b, k, buf):
        dst = out_hbm.at[b, pl.ds(k * CCH, CCH)]
        return pltpu.make_async_copy(outs[buf].reshape(CCH, GH, GW), dst,
                                     osems[buf])

    def do_chunk(b, buf):
        img_ref = imgs[buf]
        out_ref = outs[buf]
        zero = jnp.zeros((L,), jnp.int32)
        for g in range(NGRP):
            sl = pl.ds(g * L, L)
            t00 = idx_v[0, sl]
            t01 = idx_v[1, sl]
            t10 = idx_v[2, sl]
            t11 = idx_v[3, sl]
            to = idx_v[4, sl]
            w00 = wt_v[0, sl]
            w01 = wt_v[1, sl]
            w10 = wt_v[2, sl]
            w11 = wt_v[3, sl]
            mask = None
            if (g + 1) * L > NPTS:
                mask = (lax.iota(jnp.int32, L) + g * L) < NPTS
            for c in range(CCH):
                cb = c * (H * 128)
                ob = c * (GH * 128)
                v00 = plsc.load_gather(img_ref, [zero, t00 + cb])
                v01 = plsc.load_gather(img_ref, [zero, t01 + cb])
                v10 = plsc.load_gather(img_ref, [zero, t10 + cb])
                v11 = plsc.load_gather(img_ref, [zero, t11 + cb])
                res = v00 * w00 + v01 * w01 + v10 * w10 + v11 * w11
                plsc.store_scatter(out_ref, [zero, to + ob], res, mask=mask)

    def batch_body(bi, carry):
        b = wid * BPW + bi
        gcp = pltpu.make_async_copy(grid_hbm.at[b], grid_v, gsem)
        gcp.start()
        img_copy(b, 0, 0).start()
        gcp.wait()
        _tap_setup(grid_v, idx_v, wt_v)

        def chunk_pair(j, carry2):
            k0 = j * 2
            img_copy(b, k0 + 1, 1).start()

            @pl.when(j > 0)
            def _():
                out_copy(b, k0 - 2, 0).wait()

            img_copy(b, k0, 0).wait()
            do_chunk(b, 0)
            out_copy(b, k0, 0).start()
            @pl.when(j + 1 < NCHUNK // 2)
            def _():
                img_copy(b, k0 + 2, 0).start()

            @pl.when(j > 0)
            def _():
                out_copy(b, k0 - 1, 1).wait()

            img_copy(b, k0 + 1, 1).wait()
            do_chunk(b, 1)
            out_copy(b, k0 + 1, 1).start()
            return carry2

        lax.fori_loop(0, NCHUNK // 2, chunk_pair, 0)
        out_copy(b, NCHUNK - 2, 0).wait()
        out_copy(b, NCHUNK - 1, 1).wait()
        return carry

    lax.fori_loop(0, BPW, batch_body, 0)


@jax.jit
def kernel(input1, input2):
    mesh = plsc.VectorSubcoreMesh(
        core_axis_name="c", subcore_axis_name="s",
        num_cores=NCORES, num_subcores=NSUB)
    grid = jnp.transpose(input2.reshape(B, NPTS, 2), (0, 2, 1))
    grid = jnp.pad(grid, ((0, 0), (0, 0), (0, NGRP * L - NPTS)))
    return pl.kernel(
        _body,
        out_type=jax.ShapeDtypeStruct((B, C, GH, GW), jnp.float32),
        mesh=mesh,
        compiler_params=pltpu.CompilerParams(needs_layout_passes=False),
        scratch_types=[
            pltpu.VMEM((2, NGRP * L), jnp.float32),
            pltpu.VMEM((6, NGRP * L), jnp.int32),
            pltpu.VMEM((4, NGRP * L), jnp.float32),
            pltpu.VMEM((CCH * H, W), jnp.float32),
            pltpu.VMEM((CCH * H, W), jnp.float32),
            pltpu.VMEM((CCH * GH, GW), jnp.float32),
            pltpu.VMEM((CCH * GH, GW), jnp.float32),
            pltpu.SemaphoreType.DMA,
            pltpu.SemaphoreType.DMA,
            pltpu.SemaphoreType.DMA,
            pltpu.SemaphoreType.DMA,
            pltpu.SemaphoreType.DMA,
        ],
    )(input1, grid)


# speedup vs baseline: 1.5546x; 1.1553x over previous
import functools

import jax
import jax.numpy as jnp
from jax import lax
from jax.experimental import pallas as pl
from jax.experimental.pallas import tpu as pltpu
from jax.experimental.pallas import tpu_sc as plsc

B, C, H, W = 256, 256, 28, 28
GH, GW = 14, 14
NPTS = GH * GW
L = 16
NGRP = (NPTS + L - 1) // L
NCORES, NSUB = 2, 16
NW = NCORES * NSUB
BPW = B // NW
CCH = 8
NCHUNK = C // CCH


def _tap_setup(grid_v, idx_v, wt_v):
    for g in range(NGRP):
        sl = pl.ds(g * L, L)
        p = lax.iota(jnp.int32, L) + (g * L)
        p = jnp.minimum(p, NPTS - 1)
        gy = grid_v[0, sl]
        gx = grid_v[1, sl]
        y = (gy + 1.0) * ((H - 1) * 0.5)
        x = (gx + 1.0) * ((W - 1) * 0.5)
        y = jnp.minimum(jnp.maximum(y, 0.0), float(H - 1))
        x = jnp.minimum(jnp.maximum(x, 0.0), float(W - 1))
        y0 = y.astype(jnp.int32)
        x0 = x.astype(jnp.int32)
        wy1 = y - y0.astype(jnp.float32)
        wx1 = x - x0.astype(jnp.float32)
        y1 = jnp.minimum(y0 + 1, H - 1)
        x1 = jnp.minimum(x0 + 1, W - 1)
        wy0 = 1.0 - wy1
        wx0 = 1.0 - wx1
        yb0 = y0 * 128
        yb1 = y1 * 128
        idx_v[0, sl] = yb0 + x0
        idx_v[1, sl] = yb0 + x1
        idx_v[2, sl] = yb1 + x0
        idx_v[3, sl] = yb1 + x1
        idx_v[4, sl] = lax.div(p, GW) * 128 + lax.rem(p, GW)
        wt_v[0, sl] = wy0 * wx0
        wt_v[1, sl] = wy0 * wx1
        wt_v[2, sl] = wy1 * wx0
        wt_v[3, sl] = wy1 * wx1


def _body(img_hbm, grid_hbm, out_hbm, grid_v, idx_v, wt_v, img_v0, img_v1,
          out_v0, out_v1, gsem, isem0, isem1, osem0, osem1):
    cid = lax.axis_index("c")
    sid = lax.axis_index("s")
    wid = sid * NCORES + cid
    imgs = (img_v0, img_v1)
    outs = (out_v0, out_v1)
    isems = (isem0, isem1)
    osems = (osem0, osem1)

    def img_copy(b, k, buf):
        src = img_hbm.at[b, pl.ds(k * CCH, CCH)]
        return pltpu.make_async_copy(src, imgs[buf].reshape(CCH, H, W),
                                     isems[buf])

    def out_copy(b, k, buf):
        dst = out_hbm.at[b, pl.ds(k * CCH, CCH)]
        return pltpu.make_async_copy(outs[buf].reshape(CCH, GH, GW), dst,
                                     osems[buf])

    def do_chunk(b, buf):
        img_ref = imgs[buf]
        out_ref = outs[buf]
        zero = jnp.zeros((L,), jnp.int32)
        for g in range(NGRP):
            sl = pl.ds(g * L, L)
            t00 = idx_v[0, sl]
            t01 = idx_v[1, sl]
            t10 = idx_v[2, sl]
            t11 = idx_v[3, sl]
            to = idx_v[4, sl]
            w00 = wt_v[0, sl]
            w01 = wt_v[1, sl]
            w10 = wt_v[2, sl]
            w11 = wt_v[3, sl]
            mask = None
            if (g + 1) * L > NPTS:
                mask = (lax.iota(jnp.int32, L) + g * L) < NPTS
            @plsc.parallel_loop(0, CCH, unroll=CCH)
            def _ch_body(c):
                cb = c * (H * 128)
                ob = c * (GH * 128)
                v00 = plsc.load_gather(img_ref, [zero, t00 + cb])
                v01 = plsc.load_gather(img_ref, [zero, t01 + cb])
                v10 = plsc.load_gather(img_ref, [zero, t10 + cb])
                v11 = plsc.load_gather(img_ref, [zero, t11 + cb])
                res = (v00 * w00 + v01 * w01) + (v10 * w10 + v11 * w11)
                plsc.store_scatter(out_ref, [zero, to + ob], res, mask=mask)

    def batch_body(bi, carry):
        b = wid * BPW + bi
        gcp = pltpu.make_async_copy(grid_hbm.at[b], grid_v, gsem)
        gcp.start()
        img_copy(b, 0, 0).start()
        gcp.wait()
        _tap_setup(grid_v, idx_v, wt_v)

        def chunk_pair(j, carry2):
            k0 = j * 2
            img_copy(b, k0 + 1, 1).start()

            @pl.when(j > 0)
            def _():
                out_copy(b, k0 - 2, 0).wait()

            img_copy(b, k0, 0).wait()
            do_chunk(b, 0)
            out_copy(b, k0, 0).start()
            @pl.when(j + 1 < NCHUNK // 2)
            def _():
                img_copy(b, k0 + 2, 0).start()

            @pl.when(j > 0)
            def _():
                out_copy(b, k0 - 1, 1).wait()

            img_copy(b, k0 + 1, 1).wait()
            do_chunk(b, 1)
            out_copy(b, k0 + 1, 1).start()
            return carry2

        lax.fori_loop(0, NCHUNK // 2, chunk_pair, 0)
        out_copy(b, NCHUNK - 2, 0).wait()
        out_copy(b, NCHUNK - 1, 1).wait()
        return carry

    lax.fori_loop(0, BPW, batch_body, 0)


@jax.jit
def kernel(input1, input2):
    mesh = plsc.VectorSubcoreMesh(
        core_axis_name="c", subcore_axis_name="s",
        num_cores=NCORES, num_subcores=NSUB)
    grid = jnp.transpose(input2.reshape(B, NPTS, 2), (0, 2, 1))
    grid = jnp.pad(grid, ((0, 0), (0, 0), (0, NGRP * L - NPTS)))
    return pl.kernel(
        _body,
        out_type=jax.ShapeDtypeStruct((B, C, GH, GW), jnp.float32),
        mesh=mesh,
        compiler_params=pltpu.CompilerParams(needs_layout_passes=False),
        scratch_types=[
            pltpu.VMEM((2, NGRP * L), jnp.float32),
            pltpu.VMEM((6, NGRP * L), jnp.int32),
            pltpu.VMEM((4, NGRP * L), jnp.float32),
            pltpu.VMEM((CCH * H, W), jnp.float32),
            pltpu.VMEM((CCH * H, W), jnp.float32),
            pltpu.VMEM((CCH * GH, GW), jnp.float32),
            pltpu.VMEM((CCH * GH, GW), jnp.float32),
            pltpu.SemaphoreType.DMA,
            pltpu.SemaphoreType.DMA,
            pltpu.SemaphoreType.DMA,
            pltpu.SemaphoreType.DMA,
            pltpu.SemaphoreType.DMA,
        ],
    )(input1, grid)


# speedup vs baseline: 1.5562x; 1.0011x over previous
import jax
import jax.numpy as jnp
from jax import lax
from jax.experimental import pallas as pl
from jax.experimental.pallas import tpu as pltpu
from jax.experimental.pallas import tpu_sc as plsc

B, C, H, W = 256, 256, 28, 28
GH, GW = 14, 14
NPTS = GH * GW
L = 16
NGRP = (NPTS + L - 1) // L
NCORES, NSUB = 2, 16
NW = NCORES * NSUB
BPW = B // NW
CCH = 8
NCHUNK = C // CCH


def _tap_setup(grid_v, idx_v, wt_v):
    for g in range(NGRP):
        sl = pl.ds(g * L, L)
        p = lax.iota(jnp.int32, L) + (g * L)
        p = jnp.minimum(p, NPTS - 1)
        gy = grid_v[0, sl]
        gx = grid_v[1, sl]
        y = (gy + 1.0) * ((H - 1) * 0.5)
        x = (gx + 1.0) * ((W - 1) * 0.5)
        y = jnp.minimum(jnp.maximum(y, 0.0), float(H - 1))
        x = jnp.minimum(jnp.maximum(x, 0.0), float(W - 1))
        y0 = y.astype(jnp.int32)
        x0 = x.astype(jnp.int32)
        wy1 = y - y0.astype(jnp.float32)
        wx1 = x - x0.astype(jnp.float32)
        y1 = jnp.minimum(y0 + 1, H - 1)
        x1 = jnp.minimum(x0 + 1, W - 1)
        wy0 = 1.0 - wy1
        wx0 = 1.0 - wx1
        yb0 = y0 * 128
        yb1 = y1 * 128
        idx_v[0, sl] = yb0 + x0
        idx_v[1, sl] = yb0 + x1
        idx_v[2, sl] = yb1 + x0
        idx_v[3, sl] = yb1 + x1
        idx_v[4, sl] = lax.div(p, GW) * 128 + lax.rem(p, GW)
        wt_v[0, sl] = wy0 * wx0
        wt_v[1, sl] = wy0 * wx1
        wt_v[2, sl] = wy1 * wx0
        wt_v[3, sl] = wy1 * wx1


def _body(img_hbm, grid_hbm, out_hbm, grid_v, idx_v, wt_v, img_v0, img_v1,
          out_v0, out_v1, gsem, isem0, isem1, osem0, osem1):
    cid = lax.axis_index("c")
    sid = lax.axis_index("s")
    wid = sid * NCORES + cid
    imgs = (img_v0, img_v1)
    outs = (out_v0, out_v1)
    isems = (isem0, isem1)
    osems = (osem0, osem1)

    def img_copy(b, k, buf):
        src = img_hbm.at[b, pl.ds(k * CCH, CCH)]
        return pltpu.make_async_copy(src, imgs[buf].reshape(CCH, H, W),
                                     isems[buf])

    def out_copy(b, k, buf):
        dst = out_hbm.at[b, pl.ds(k * CCH, CCH)]
        return pltpu.make_async_copy(outs[buf].reshape(CCH, GH, GW), dst,
                                     osems[buf])

    def do_chunk(b, buf):
        img_ref = imgs[buf]
        out_ref = outs[buf]
        zero = jnp.zeros((L,), jnp.int32)
        for g in range(NGRP):
            sl = pl.ds(g * L, L)
            t00 = idx_v[0, sl]
            t01 = idx_v[1, sl]
            t10 = idx_v[2, sl]
            t11 = idx_v[3, sl]
            to = idx_v[4, sl]
            w00 = wt_v[0, sl]
            w01 = wt_v[1, sl]
            w10 = wt_v[2, sl]
            w11 = wt_v[3, sl]
            mask = None
            if (g + 1) * L > NPTS:
                mask = (lax.iota(jnp.int32, L) + g * L) < NPTS
            @plsc.parallel_loop(0, CCH, unroll=CCH)
            def _ch_body(c):
                cb = c * (H * 128)
                ob = c * (GH * 128)
                v00 = plsc.load_gather(img_ref, [zero, t00 + cb])
                v01 = plsc.load_gather(img_ref, [zero, t01 + cb])
                v10 = plsc.load_gather(img_ref, [zero, t10 + cb])
                v11 = plsc.load_gather(img_ref, [zero, t11 + cb])
                res = (v00 * w00 + v01 * w01) + (v10 * w10 + v11 * w11)
                plsc.store_scatter(out_ref, [zero, to + ob], res, mask=mask)

    def batch_body(bi, carry):
        b = wid * BPW + bi
        gcp = pltpu.make_async_copy(grid_hbm.at[b], grid_v, gsem)
        gcp.start()
        img_copy(b, 0, 0).start()
        gcp.wait()
        _tap_setup(grid_v, idx_v, wt_v)

        def chunk_pair(j, carry2):
            k0 = j * 2
            img_copy(b, k0 + 1, 1).start()

            @pl.when(j > 0)
            def _():
                out_copy(b, k0 - 2, 0).wait()

            img_copy(b, k0, 0).wait()
            do_chunk(b, 0)
            out_copy(b, k0, 0).start()
            @pl.when(j + 1 < NCHUNK // 2)
            def _():
                img_copy(b, k0 + 2, 0).start()

            @pl.when(j > 0)
            def _():
                out_copy(b, k0 - 1, 1).wait()

            img_copy(b, k0 + 1, 1).wait()
            do_chunk(b, 1)
            out_copy(b, k0 + 1, 1).start()
            return carry2

        lax.fori_loop(0, NCHUNK // 2, chunk_pair, 0)
        out_copy(b, NCHUNK - 2, 0).wait()
        out_copy(b, NCHUNK - 1, 1).wait()
        return carry

    lax.fori_loop(0, BPW, batch_body, 0)


@jax.jit
def kernel(input1, input2):
    mesh = plsc.VectorSubcoreMesh(
        core_axis_name="c", subcore_axis_name="s",
        num_cores=NCORES, num_subcores=NSUB)
    grid = jnp.transpose(input2.reshape(B, NPTS, 2), (0, 2, 1))
    grid = jnp.pad(grid, ((0, 0), (0, 0), (0, NGRP * L - NPTS)))
    return pl.kernel(
        _body,
        out_type=jax.ShapeDtypeStruct((B, C, GH, GW), jnp.float32),
        mesh=mesh,
        compiler_params=pltpu.CompilerParams(needs_layout_passes=False),
        scratch_types=[
            pltpu.VMEM((2, NGRP * L), jnp.float32),
            pltpu.VMEM((6, NGRP * L), jnp.int32),
            pltpu.VMEM((4, NGRP * L), jnp.float32),
            pltpu.VMEM((CCH * H, W), jnp.float32),
            pltpu.VMEM((CCH * H, W), jnp.float32),
            pltpu.VMEM((CCH * GH, GW), jnp.float32),
            pltpu.VMEM((CCH * GH, GW), jnp.float32),
            pltpu.SemaphoreType.DMA,
            pltpu.SemaphoreType.DMA,
            pltpu.SemaphoreType.DMA,
            pltpu.SemaphoreType.DMA,
            pltpu.SemaphoreType.DMA,
        ],
    )(input1, grid)
